# Initial kernel scaffold; baseline (speedup 1.0000x reference)
import jax
import jax.numpy as jnp
from jax.experimental import pallas as pl


def kernel(text, token_embedding, pos_embed):
    raise NotImplementedError("write your pallas kernel here")


# speedup vs baseline: 2.0668x; 2.0668x over previous
import jax
import jax.numpy as jnp
from jax import lax
from jax.experimental import pallas as pl
from jax.experimental.pallas import tpu as pltpu
from jax.experimental.pallas import tpu_sc as plsc

VOCAB = 100000
CTX = 200
DIM = 64
BATCH = 1024

NC = 2
NS = 16
NW = NC * NS
SEQ_PER_W = BATCH // NW
ROWS_PER_W = SEQ_PER_W * CTX
CH0 = 128
CH1 = CTX - CH0
LANES = 16


def _sc_body(table_hbm, text_hbm, text_t_hbm, pos_hbm, out_hbm, len_hbm,
             idx_v, idxt_v, pos_v, rows_v, len_v, sem):
    wid = lax.axis_index("s") * NC + lax.axis_index("c")
    base_row = wid * ROWS_PER_W
    base_seq = wid * SEQ_PER_W

    pltpu.sync_copy(text_hbm.at[pl.ds(base_row, ROWS_PER_W)], idx_v)
    pltpu.sync_copy(text_t_hbm.at[pl.ds(base_row, ROWS_PER_W)], idxt_v)
    pltpu.sync_copy(pos_hbm, pos_v)

    def am_body(p, accs):
        a0, a1 = accs
        v0 = idxt_v[pl.ds(p * SEQ_PER_W, LANES)]
        v1 = idxt_v[pl.ds(p * SEQ_PER_W + LANES, LANES)]
        k = 255 - p
        return (jnp.maximum(a0, v0 * 256 + k), jnp.maximum(a1, v1 * 256 + k))

    init = jnp.full((LANES,), -1, jnp.int32)
    a0, a1 = lax.fori_loop(0, CTX, am_body, (init, init), unroll=4)
    len_v[pl.ds(0, LANES)] = 255 - (a0 & 255)
    len_v[pl.ds(LANES, LANES)] = 255 - (a1 & 255)
    pltpu.sync_copy(len_v, len_hbm.at[pl.ds(base_seq, SEQ_PER_W)])

    def seq_body(s, carry):
        cp0 = pltpu.async_copy(
            table_hbm.at[idx_v.at[pl.ds(s * CTX, CH0)]],
            rows_v.at[pl.ds(0, CH0)], sem)
        cp1 = pltpu.async_copy(
            table_hbm.at[idx_v.at[pl.ds(s * CTX + CH0, CH1)]],
            rows_v.at[pl.ds(CH0, CH1)], sem)
        cp0.wait()
        cp1.wait()

        def add_body(r, c):
            for j in range(DIM // LANES):
                sl = pl.ds(j * LANES, LANES)
                rows_v[r, sl] = rows_v[r, sl] + pos_v[r, sl]
            return c
        lax.fori_loop(0, CTX, add_body, 0, unroll=2)

        pltpu.sync_copy(rows_v, out_hbm.at[pl.ds(base_row + s * CTX, CTX)])
        return carry

    lax.fori_loop(0, SEQ_PER_W, seq_body, 0)


def kernel(text, token_embedding, pos_embed):
    text = text.astype(jnp.int32)
    text_flat = text.reshape(BATCH * CTX)
    text_t = (text.reshape(NW, SEQ_PER_W, CTX)
              .transpose(0, 2, 1).reshape(NW * CTX * SEQ_PER_W))
    pos2d = pos_embed.reshape(CTX, DIM).astype(jnp.float32)
    table = token_embedding.astype(jnp.float32)

    mesh = plsc.VectorSubcoreMesh(core_axis_name="c", subcore_axis_name="s")
    f = pl.kernel(
        _sc_body,
        mesh=mesh,
        compiler_params=pltpu.CompilerParams(use_tc_tiling_on_sc=False),
        out_type=[
            jax.ShapeDtypeStruct((BATCH * CTX, DIM), jnp.float32),
            jax.ShapeDtypeStruct((BATCH,), jnp.int32),
        ],
        scratch_types=[
            pltpu.VMEM((ROWS_PER_W,), jnp.int32),
            pltpu.VMEM((ROWS_PER_W,), jnp.int32),
            pltpu.VMEM((CTX, DIM), jnp.float32),
            pltpu.VMEM((CTX, DIM), jnp.float32),
            pltpu.VMEM((SEQ_PER_W,), jnp.int32),
            pltpu.SemaphoreType.DMA,
        ],
    )
    rows, lengths = f(table, text_flat, text_t, pos2d)
    return (rows.reshape(BATCH, CTX, DIM), lengths)
